# baseline (speedup 1.0000x reference)
import functools

import jax
import jax.numpy as jnp
from jax import lax
from jax.experimental import pallas as pl
from jax.experimental.pallas import tpu as pltpu
from jax.experimental.pallas import tpu_sc as plsc

N = 10000
E = 320000
IN_CH = 128
HID = 128
OUT = 64

NC = 2
NS = 16
NW = NC * NS
RPT = 640
NPAD = NS * RPT


def _nch(k):
  n = -(-E // (NW * k))
  return n + (n % 2)

EPAD = NW * 128 * _nch(128)


def _make_seg(D, K, with_cnt):
  NCH = _nch(K)
  mesh = plsc.VectorSubcoreMesh(core_axis_name="c", subcore_axis_name="s")
  out_type = [jax.ShapeDtypeStruct((NC, NPAD, D), jnp.float32)]
  scratch = [
      pltpu.VMEM((NCH, K), jnp.int32),
      pltpu.VMEM((K,), jnp.int32),
      pltpu.VMEM((K,), jnp.int32),
      pltpu.VMEM((K, D), jnp.float32),
      pltpu.VMEM((K, D), jnp.float32),
      pltpu.VMEM_SHARED((NPAD, D), jnp.float32),
      pltpu.SemaphoreType.DMA,
      pltpu.SemaphoreType.DMA,
  ]
  if with_cnt:
    out_type.append(jax.ShapeDtypeStruct((NC, NS, RPT), jnp.float32))
    scratch += [
        pltpu.VMEM_SHARED((NPAD,), jnp.float32),
        pltpu.VMEM((RPT,), jnp.float32),
        pltpu.VMEM((K,), jnp.float32),
    ]

  def body(p_hbm, src_hbm, dst_hbm, *rest):
    if with_cnt:
      (acc_out, cnt_out, src_v, dst0_v, dst1_v, rows0_v, rows1_v, acc_sh,
       sem0, sem1, cnt_sh, zrow_v, ones_v) = rest
    else:
      (acc_out, src_v, dst0_v, dst1_v, rows0_v, rows1_v, acc_sh,
       sem0, sem1) = rest
    sems = (sem0, sem1)
    rows = (rows0_v, rows1_v)
    dsts = (dst0_v, dst1_v)
    cid = lax.axis_index("c")
    sid = lax.axis_index("s")
    wid = cid * NS + sid
    r0 = sid * RPT

    zv = jnp.zeros((16,), jnp.float32)

    def zb(i, carry):
      for l in range(D // 16):
        rows0_v[i, pl.ds(l * 16, 16)] = zv
      return carry
    lax.fori_loop(0, K, zb, 0)
    if with_cnt:
      def cb(i, carry):
        zrow_v[pl.ds(i * 16, 16)] = zv
        return carry
      lax.fori_loop(0, RPT // 16, cb, 0)
      def ob(i, carry):
        ones_v[pl.ds(i * 16, 16)] = zv + 1.0
        return carry
      lax.fori_loop(0, K // 16, ob, 0)
      pltpu.sync_copy(zrow_v, cnt_sh.at[pl.ds(r0, RPT)])

    off = 0
    while off < RPT:
      seg = min(K, RPT - off)
      pltpu.sync_copy(rows0_v.at[pl.ds(0, seg)],
                      acc_sh.at[pl.ds(r0 + off, seg)])
      off += seg
    pltpu.sync_copy(src_hbm.at[wid], src_v)
    plsc.subcore_barrier()

    for b in range(2):
      pltpu.async_copy(p_hbm.at[src_v.at[b]], rows[b], sems[b])
      pltpu.async_copy(dst_hbm.at[wid, b], dsts[b], sems[b])

    def wait_pair(b):
      pltpu.make_async_copy(p_hbm.at[pl.ds(0, K)], rows[b], sems[b]).wait()
      pltpu.make_async_copy(dst_hbm.at[0, 0], dsts[b], sems[b]).wait()

    def eb(i, carry):
      for b in range(2):
        j = i * 2 + b
        wait_pair(b)
        pltpu.sync_copy(rows[b], acc_sh.at[dsts[b]], add=True)
        if with_cnt:
          pltpu.sync_copy(ones_v, cnt_sh.at[dsts[b]], add=True)
        jn = jnp.minimum(j + 2, NCH - 1)
        pltpu.async_copy(p_hbm.at[src_v.at[jn]], rows[b], sems[b])
        pltpu.async_copy(dst_hbm.at[wid, jn], dsts[b], sems[b])
      return carry
    lax.fori_loop(0, NCH // 2, eb, 0)
    for b in range(2):
      wait_pair(b)
    plsc.subcore_barrier()

    pltpu.sync_copy(acc_sh.at[pl.ds(r0, RPT)], acc_out.at[cid, pl.ds(r0, RPT)])
    if with_cnt:
      pltpu.sync_copy(cnt_sh.at[pl.ds(r0, RPT)], cnt_out.at[cid, sid])

  return pl.kernel(body, out_type=tuple(out_type), mesh=mesh,
                   scratch_types=tuple(scratch),
                   compiler_params=pltpu.CompilerParams(
                       use_tc_tiling_on_sc=False))


K1 = 128
K2 = 256
_seg_cnt = _make_seg(HID, K1, True)
_seg2 = _make_seg(OUT, K2, False)


def _mm1_body(x_ref, wl_ref, wr_ref, b_ref, p_ref, r_ref):
  x = x_ref[...]
  p_ref[...] = jnp.dot(x, wl_ref[...], preferred_element_type=jnp.float32)
  r_ref[...] = jnp.dot(x, wr_ref[...],
                       preferred_element_type=jnp.float32) + b_ref[...]


_mm1 = pl.pallas_call(
    _mm1_body,
    out_shape=(jax.ShapeDtypeStruct((N, HID), jnp.float32),
               jax.ShapeDtypeStruct((N, HID), jnp.float32)),
)


def _mid_body(acc_ref, cnt_ref, r1_ref, wl_ref, wr_ref, b_ref, p2_ref, r2_ref):
  agg = acc_ref[0, :N, :] + acc_ref[1, :N, :]
  cnt = jnp.sum(cnt_ref[:, :N], axis=0)[:, None]
  rc = 1.0 / jnp.maximum(cnt, 1.0)
  h = jnp.maximum(agg * rc + r1_ref[...], 0.0)
  p2_ref[...] = jnp.dot(h, wl_ref[...], preferred_element_type=jnp.float32)
  r2_ref[...] = jnp.dot(h, wr_ref[...],
                        preferred_element_type=jnp.float32) + b_ref[...]


_mid = pl.pallas_call(
    _mid_body,
    out_shape=(jax.ShapeDtypeStruct((N, OUT), jnp.float32),
               jax.ShapeDtypeStruct((N, OUT), jnp.float32)),
)


def _fin_body(acc_ref, cnt_ref, r2_ref, o_ref):
  agg = acc_ref[0, :N, :] + acc_ref[1, :N, :]
  cnt = jnp.sum(cnt_ref[:, :N], axis=0)[:, None]
  o = agg * (1.0 / jnp.maximum(cnt, 1.0)) + r2_ref[...]
  m = jnp.max(o, axis=-1, keepdims=True)
  o_ref[...] = (o - m) - jnp.log(jnp.sum(jnp.exp(o - m), axis=-1,
                                         keepdims=True))


_fin = pl.pallas_call(
    _fin_body,
    out_shape=jax.ShapeDtypeStruct((N, OUT), jnp.float32),
)


def kernel(x, edge_index, W1l, W1r, b1, W2l, W2r, b2):
  src = edge_index[0]
  dst = edge_index[1]
  pad = EPAD - E
  srcp = jnp.concatenate([src, jnp.zeros((pad,), jnp.int32)])
  dstp = jnp.concatenate([dst, jnp.full((pad,), N, jnp.int32)])
  src1 = srcp.reshape(NW, _nch(K1), K1)
  dst1 = dstp.reshape(NW, _nch(K1), K1)
  src2 = srcp.reshape(NW, _nch(K2), K2)
  dst2 = dstp.reshape(NW, _nch(K2), K2)
  p1, r1 = _mm1(x, W1l, W1r, b1.reshape(1, HID))
  acc1, cnt = _seg_cnt(p1, src1, dst1)
  cnt = cnt.reshape(NC, NPAD)
  p2, r2 = _mid(acc1, cnt, r1, W2l, W2r, b2.reshape(1, OUT))
  acc2 = _seg2(p2, src2, dst2)[0]
  return _fin(acc2, cnt, r2)


# speedup vs baseline: 1.2306x; 1.2306x over previous
import functools

import jax
import jax.numpy as jnp
from jax import lax
from jax.experimental import pallas as pl
from jax.experimental.pallas import tpu as pltpu
from jax.experimental.pallas import tpu_sc as plsc

N = 10000
E = 320000
IN_CH = 128
HID = 128
OUT = 64

NC = 2
NS = 16
RPT = 632
NPAD = NS * RPT

C0 = 245760
C1 = E - C0


def _nch(edges, k):
  n = -(-edges // (NS * k))
  return n + (n % 2)


def _make_seg(D, K, with_cnt):
  NCH0 = _nch(C0, K)
  NCH1 = _nch(C1, K)
  mesh = plsc.VectorSubcoreMesh(core_axis_name="c", subcore_axis_name="s")
  out_type = [jax.ShapeDtypeStruct((NC, NPAD, D), jnp.float32)]
  scratch = [
      pltpu.VMEM((NCH0, K), jnp.int32),
      pltpu.VMEM((K,), jnp.int32),
      pltpu.VMEM((K,), jnp.int32),
      pltpu.VMEM((K, D), jnp.float32),
      pltpu.VMEM((K, D), jnp.float32),
      pltpu.VMEM_SHARED((NPAD, D), jnp.float32),
      pltpu.SemaphoreType.DMA,
      pltpu.SemaphoreType.DMA,
  ]
  if with_cnt:
    out_type.append(jax.ShapeDtypeStruct((NC, NS, RPT), jnp.float32))
    scratch += [
        pltpu.VMEM_SHARED((NPAD,), jnp.float32),
        pltpu.VMEM((640,), jnp.float32),
        pltpu.VMEM((K,), jnp.float32),
    ]

  def body(p_hbm, src0_hbm, dst0_hbm, src1_hbm, dst1_hbm, *rest):
    if with_cnt:
      (acc_out, cnt_out, src_v, dst0_v, dst1_v, rows0_v, rows1_v, acc_sh,
       sem0, sem1, cnt_sh, zrow_v, ones_v) = rest
    else:
      (acc_out, src_v, dst0_v, dst1_v, rows0_v, rows1_v, acc_sh,
       sem0, sem1) = rest
    sems = (sem0, sem1)
    rows = (rows0_v, rows1_v)
    dsts = (dst0_v, dst1_v)
    cid = lax.axis_index("c")
    sid = lax.axis_index("s")
    r0 = sid * RPT

    zv = jnp.zeros((16,), jnp.float32)

    def zb(i, carry):
      for l in range(D // 16):
        rows0_v[i, pl.ds(l * 16, 16)] = zv
      return carry
    lax.fori_loop(0, K, zb, 0)
    if with_cnt:
      def cb(i, carry):
        zrow_v[pl.ds(i * 16, 16)] = zv
        return carry
      lax.fori_loop(0, 640 // 16, cb, 0)
      def ob(i, carry):
        ones_v[pl.ds(i * 16, 16)] = zv + 1.0
        return carry
      lax.fori_loop(0, K // 16, ob, 0)
      pltpu.sync_copy(zrow_v.at[pl.ds(0, RPT)], cnt_sh.at[pl.ds(r0, RPT)])

    off = 0
    while off < RPT:
      seg = min(K, RPT - off)
      pltpu.sync_copy(rows0_v.at[pl.ds(0, seg)],
                      acc_sh.at[pl.ds(r0 + off, seg)])
      off += seg
    plsc.subcore_barrier()

    def run_edges(src_hbm, dst_hbm, nch):
      pltpu.sync_copy(src_hbm.at[sid], src_v.at[pl.ds(0, nch)])
      for b in range(2):
        pltpu.async_copy(p_hbm.at[src_v.at[b]], rows[b], sems[b])
        pltpu.async_copy(dst_hbm.at[sid, b], dsts[b], sems[b])

      def wait_pair(b):
        pltpu.make_async_copy(p_hbm.at[pl.ds(0, K)], rows[b], sems[b]).wait()
        pltpu.make_async_copy(dst_hbm.at[0, 0], dsts[b], sems[b]).wait()

      def eb(i, carry):
        for b in range(2):
          j = i * 2 + b
          wait_pair(b)
          pltpu.sync_copy(rows[b], acc_sh.at[dsts[b]], add=True)
          if with_cnt:
            pltpu.sync_copy(ones_v, cnt_sh.at[dsts[b]], add=True)
          jn = jnp.minimum(j + 2, nch - 1)
          pltpu.async_copy(p_hbm.at[src_v.at[jn]], rows[b], sems[b])
          pltpu.async_copy(dst_hbm.at[sid, jn], dsts[b], sems[b])
        return carry
      lax.fori_loop(0, nch // 2, eb, 0)
      for b in range(2):
        wait_pair(b)

    @pl.when(cid == 0)
    def _():
      run_edges(src0_hbm, dst0_hbm, NCH0)

    @pl.when(cid != 0)
    def _():
      run_edges(src1_hbm, dst1_hbm, NCH1)

    plsc.subcore_barrier()

    pltpu.sync_copy(acc_sh.at[pl.ds(r0, RPT)], acc_out.at[cid, pl.ds(r0, RPT)])
    if with_cnt:
      pltpu.sync_copy(cnt_sh.at[pl.ds(r0, RPT)], cnt_out.at[cid, sid])

  return pl.kernel(body, out_type=tuple(out_type), mesh=mesh,
                   scratch_types=tuple(scratch),
                   compiler_params=pltpu.CompilerParams(
                       use_tc_tiling_on_sc=False))


K1 = 128
K2 = 256
_seg_cnt = _make_seg(HID, K1, True)
_seg2 = _make_seg(OUT, K2, False)


def _mm1_body(x_ref, wl_ref, wr_ref, b_ref, p_ref, r_ref):
  x = x_ref[...]
  p_ref[...] = jnp.dot(x, wl_ref[...], preferred_element_type=jnp.float32)
  r_ref[...] = jnp.dot(x, wr_ref[...],
                       preferred_element_type=jnp.float32) + b_ref[...]


_mm1 = pl.pallas_call(
    _mm1_body,
    out_shape=(jax.ShapeDtypeStruct((N, HID), jnp.float32),
               jax.ShapeDtypeStruct((N, HID), jnp.float32)),
)


def _mid_body(acc_ref, cnt_ref, r1_ref, wl_ref, wr_ref, b_ref, p2_ref, r2_ref):
  agg = acc_ref[0, :N, :] + acc_ref[1, :N, :]
  cnt = jnp.sum(cnt_ref[:, :N], axis=0)[:, None]
  rc = 1.0 / jnp.maximum(cnt, 1.0)
  h = jnp.maximum(agg * rc + r1_ref[...], 0.0)
  p2_ref[...] = jnp.dot(h, wl_ref[...], preferred_element_type=jnp.float32)
  r2_ref[...] = jnp.dot(h, wr_ref[...],
                        preferred_element_type=jnp.float32) + b_ref[...]


_mid = pl.pallas_call(
    _mid_body,
    out_shape=(jax.ShapeDtypeStruct((N, OUT), jnp.float32),
               jax.ShapeDtypeStruct((N, OUT), jnp.float32)),
)


def _fin_body(acc_ref, cnt_ref, r2_ref, o_ref):
  agg = acc_ref[0, :N, :] + acc_ref[1, :N, :]
  cnt = jnp.sum(cnt_ref[:, :N], axis=0)[:, None]
  o = agg * (1.0 / jnp.maximum(cnt, 1.0)) + r2_ref[...]
  m = jnp.max(o, axis=-1, keepdims=True)
  o_ref[...] = (o - m) - jnp.log(jnp.sum(jnp.exp(o - m), axis=-1,
                                         keepdims=True))


_fin = pl.pallas_call(
    _fin_body,
    out_shape=jax.ShapeDtypeStruct((N, OUT), jnp.float32),
)


def _split(idx, k, fill):
  a = idx[:C0]
  b = idx[C0:]
  n0 = _nch(C0, k)
  n1 = _nch(C1, k)
  a = jnp.concatenate([a, jnp.full((NS * n0 * k - C0,), fill, jnp.int32)])
  b = jnp.concatenate([b, jnp.full((NS * n1 * k - C1,), fill, jnp.int32)])
  return a.reshape(NS, n0, k), b.reshape(NS, n1, k)


def kernel(x, edge_index, W1l, W1r, b1, W2l, W2r, b2):
  src = edge_index[0]
  dst = edge_index[1]
  s10, s11 = _split(src, K1, 0)
  d10, d11 = _split(dst, K1, N)
  s20, s21 = _split(src, K2, 0)
  d20, d21 = _split(dst, K2, N)
  p1, r1 = _mm1(x, W1l, W1r, b1.reshape(1, HID))
  acc1, cnt = _seg_cnt(p1, s10, d10, s11, d11)
  cnt = cnt.reshape(NC, NPAD)
  p2, r2 = _mid(acc1, cnt, r1, W2l, W2r, b2.reshape(1, OUT))
  acc2 = _seg2(p2, s20, d20, s21, d21)[0]
  return _fin(acc2, cnt, r2)
